# baseline (speedup 1.0000x reference)
import functools

import jax
import jax.numpy as jnp
from jax import lax
from jax.experimental import pallas as pl
from jax.experimental.pallas import tpu as pltpu
from jax.experimental.pallas import tpu_sc as plsc

_N = 10000
_NPAD = 10240
_E = 320000
_D = 128
_CHUNK = 128
_NTILES = 32
_CPT = 80
_EPT = _CHUNK * _CPT
_EPAD = _NTILES * _EPT
_CPT0 = 152
_CPT1 = 8
_SEG = 56
_SEGS0 = (56, 56, 40)
_SEGS1 = (8, 0, 0)
_RPT = _NPAD // 16
_DEGW = 8
_NBUF = 5
_SLACK = 8
_ECHUNKS = _EPAD // _CHUNK + _SLACK

_mesh = plsc.VectorSubcoreMesh(core_axis_name="c", subcore_axis_name="s")


@functools.partial(
    pl.kernel,
    out_type=jax.ShapeDtypeStruct((2 * _NPAD, _D), jnp.float32),
    mesh=_mesh,
    scratch_types=[
        pltpu.VMEM((_SEG, _CHUNK), jnp.int32),
        pltpu.VMEM((_SEG, _CHUNK), jnp.int32),
        [pltpu.VMEM((_CHUNK, _D), jnp.float32)] * 2,
        [pltpu.SemaphoreType.DMA] * 2,
        pltpu.VMEM_SHARED((_NPAD, _D), jnp.float32),
    ],
)
def _sc_spmm(h_hbm, src_hbm, dst_hbm, zeros_hbm, out_hbm, src_all, dst_all,
             rows, gsem, shared_acc):
    c = lax.axis_index("c")
    s = lax.axis_index("s")

    pltpu.sync_copy(zeros_hbm, rows[0])
    for j in range(_RPT // _CHUNK):
        pltpu.sync_copy(rows[0],
                        shared_acc.at[pl.ds(s * _RPT + j * _CHUNK, _CHUNK)])
    plsc.subcore_barrier()

    def body(t, carry):
        i0 = t * 2
        i1 = t * 2 + 1
        pltpu.async_copy(h_hbm.at[src_all.at[i0]], rows[0], gsem[0]).wait()
        pltpu.async_copy(rows[0], shared_acc.at[dst_all.at[i0]], gsem[1],
                         add=True)
        pltpu.async_copy(h_hbm.at[src_all.at[i1]], rows[1], gsem[0]).wait()
        pltpu.make_async_copy(rows[0], shared_acc.at[dst_all.at[i0]],
                              gsem[1]).wait()
        pltpu.sync_copy(rows[1], shared_acc.at[dst_all.at[i1]], add=True)
        return carry

    tile_base = jnp.where(c == 0, s * _CPT0, 16 * _CPT0 + s * _CPT1)
    seg0 = 0
    for k in range(len(_SEGS0)):
        scnt = jnp.where(c == 0, _SEGS0[k], _SEGS1[k])
        off = jnp.where(c == 0, seg0, _SEGS1[:k] and sum(_SEGS1[:k]) or 0)
        base = tile_base + off
        pltpu.sync_copy(src_hbm.at[pl.ds(base, _SEG)], src_all)
        pltpu.sync_copy(dst_hbm.at[pl.ds(base, _SEG)], dst_all)
        lax.fori_loop(0, scnt // 2, body, 0)
        seg0 += _SEGS0[k]
    plsc.subcore_barrier()
    for j in range(_RPT // _CHUNK):
        r0 = s * _RPT + j * _CHUNK
        pltpu.sync_copy(shared_acc.at[pl.ds(r0, _CHUNK)], rows[0])
        pltpu.sync_copy(rows[0], out_hbm.at[pl.ds(c * _NPAD + r0, _CHUNK)])


_BR = 1024


def _dinv_of(d0, d1):
    return lax.rsqrt(d0[:, 0:1] + d1[:, 0:1] + 1.0)


def _mm1_body(x_ref, w_ref, d0_ref, d1_ref, o_ref):
    dinv = _dinv_of(d0_ref[...], d1_ref[...])
    o_ref[...] = dinv * jnp.dot(x_ref[...], w_ref[...],
                                preferred_element_type=jnp.float32)


def _mm2_body(p0_ref, p1_ref, h1s_ref, d0_ref, d1_ref, w_ref, o_ref):
    dinv = _dinv_of(d0_ref[...], d1_ref[...])
    agg = dinv * (p0_ref[...] + p1_ref[...] + h1s_ref[...])
    h = jnp.maximum(agg, 0.0)
    o_ref[...] = dinv * jnp.dot(h, w_ref[...],
                                preferred_element_type=jnp.float32)


def _final_body(q0_ref, q1_ref, h2s_ref, d0_ref, d1_ref, o_ref):
    dinv = _dinv_of(d0_ref[...], d1_ref[...])
    agg = dinv * (q0_ref[...] + q1_ref[...] + h2s_ref[...])
    m = jnp.max(agg, axis=-1, keepdims=True)
    z = agg - m
    lse = jnp.log(jnp.sum(jnp.exp(z), axis=-1, keepdims=True))
    o_ref[...] = z - lse


def _row_spec(w):
    return pl.BlockSpec((_BR, w), lambda i: (i, 0))


def _full_spec(r, c):
    return pl.BlockSpec((r, c), lambda i: (0, 0))


def _mm1(x_pad, W1, d0, d1):
    return pl.pallas_call(
        _mm1_body,
        grid=(_NPAD // _BR,),
        in_specs=[_row_spec(_D), _full_spec(_D, _D), _row_spec(_D),
                  _row_spec(_D)],
        out_specs=_row_spec(_D),
        out_shape=jax.ShapeDtypeStruct((_NPAD, _D), jnp.float32),
    )(x_pad, W1, d0, d1)


def _mm2(p0, p1, h1s, d0, d1, W2):
    return pl.pallas_call(
        _mm2_body,
        grid=(_NPAD // _BR,),
        in_specs=[_row_spec(_D), _row_spec(_D), _row_spec(_D),
                  _row_spec(_D), _row_spec(_D), _full_spec(_D, _D)],
        out_specs=_row_spec(_D),
        out_shape=jax.ShapeDtypeStruct((_NPAD, _D), jnp.float32),
    )(p0, p1, h1s, d0, d1, W2)


def _final(q0, q1, h2s, d0, d1):
    return pl.pallas_call(
        _final_body,
        grid=(_NPAD // _BR,),
        in_specs=[_row_spec(_D), _row_spec(_D), _row_spec(_D),
                  _row_spec(_D), _row_spec(_D)],
        out_specs=_row_spec(_D),
        out_shape=jax.ShapeDtypeStruct((_NPAD, _D), jnp.float32),
    )(q0, q1, h2s, d0, d1)


def kernel(x, edge_index, W1, W2):
    src = edge_index[0]
    dst = edge_index[1]
    npadE = _ECHUNKS * _CHUNK - _E
    srcp = jnp.concatenate([src, jnp.zeros((npadE,), jnp.int32)])
    dst_flat = jnp.concatenate([dst, jnp.full((npadE,), _NPAD - 1, jnp.int32)])
    srcp = srcp.reshape(_ECHUNKS, _CHUNK)
    dstp = dst_flat.reshape(_ECHUNKS, _CHUNK)
    x_pad = jnp.pad(x, ((0, _NPAD - _N), (0, 0)))
    zeros128 = jnp.zeros((_CHUNK, _D), jnp.float32)
    ones_tab = jnp.ones((_NPAD, _D), jnp.float32)

    DP = _sc_spmm(ones_tab, srcp, dstp, zeros128)
    d0 = DP[:_NPAD]
    d1 = DP[_NPAD:]

    h1s = _mm1(x_pad, W1, d0, d1)
    P = _sc_spmm(h1s, srcp, dstp, zeros128)
    h2s = _mm2(P[:_NPAD], P[_NPAD:], h1s, d0, d1, W2)
    Q = _sc_spmm(h2s, srcp, dstp, zeros128)
    out = _final(Q[:_NPAD], Q[_NPAD:], h2s, d0, d1)
    return out[:_N]


# speedup vs baseline: 1.0054x; 1.0054x over previous
import functools

import jax
import jax.numpy as jnp
from jax import lax
from jax.experimental import pallas as pl
from jax.experimental.pallas import tpu as pltpu
from jax.experimental.pallas import tpu_sc as plsc

_N = 10000
_NPAD = 10240
_E = 320000
_D = 128
_CHUNK = 128
_NTILES = 32
_CPT = 80
_EPT = _CHUNK * _CPT
_EPAD = _NTILES * _EPT
_CPT0 = 128
_CPT1 = 32
_SEG = 56
_SEGS0 = (56, 56, 16)
_SEGS1 = (32, 0, 0)
_RPT = _NPAD // 16
_DEGW = 8
_NBUF = 5
_SLACK = 8
_ECHUNKS = _EPAD // _CHUNK + _SLACK

_mesh = plsc.VectorSubcoreMesh(core_axis_name="c", subcore_axis_name="s")


@functools.partial(
    pl.kernel,
    out_type=jax.ShapeDtypeStruct((2 * _NPAD, _D), jnp.float32),
    mesh=_mesh,
    scratch_types=[
        pltpu.VMEM((_SEG, _CHUNK), jnp.int32),
        pltpu.VMEM((_SEG, _CHUNK), jnp.int32),
        [pltpu.VMEM((_CHUNK, _D), jnp.float32)] * 2,
        [pltpu.SemaphoreType.DMA] * 2,
        pltpu.VMEM_SHARED((_NPAD, _D), jnp.float32),
    ],
)
def _sc_spmm(h_hbm, src_hbm, dst_hbm, zeros_hbm, out_hbm, src_all, dst_all,
             rows, gsem, shared_acc):
    c = lax.axis_index("c")
    s = lax.axis_index("s")

    pltpu.sync_copy(zeros_hbm, rows[0])
    for j in range(_RPT // _CHUNK):
        pltpu.sync_copy(rows[0],
                        shared_acc.at[pl.ds(s * _RPT + j * _CHUNK, _CHUNK)])
    plsc.subcore_barrier()

    def body(t, carry):
        i0 = t * 2
        i1 = t * 2 + 1
        pltpu.async_copy(h_hbm.at[src_all.at[i0]], rows[0], gsem[0]).wait()
        pltpu.async_copy(rows[0], shared_acc.at[dst_all.at[i0]], gsem[1],
                         add=True)
        pltpu.async_copy(h_hbm.at[src_all.at[i1]], rows[1], gsem[0]).wait()
        pltpu.make_async_copy(rows[0], shared_acc.at[dst_all.at[i0]],
                              gsem[1]).wait()
        pltpu.sync_copy(rows[1], shared_acc.at[dst_all.at[i1]], add=True)
        return carry

    tile_base = jnp.where(c == 0, s * _CPT0, 16 * _CPT0 + s * _CPT1)
    seg0 = 0
    for k in range(len(_SEGS0)):
        scnt = jnp.where(c == 0, _SEGS0[k], _SEGS1[k])
        off = jnp.where(c == 0, seg0, _SEGS1[:k] and sum(_SEGS1[:k]) or 0)
        base = tile_base + off
        pltpu.sync_copy(src_hbm.at[pl.ds(base, _SEG)], src_all)
        pltpu.sync_copy(dst_hbm.at[pl.ds(base, _SEG)], dst_all)
        lax.fori_loop(0, scnt // 2, body, 0)
        seg0 += _SEGS0[k]
    plsc.subcore_barrier()
    for j in range(_RPT // _CHUNK):
        r0 = s * _RPT + j * _CHUNK
        pltpu.sync_copy(shared_acc.at[pl.ds(r0, _CHUNK)], rows[0])
        pltpu.sync_copy(rows[0], out_hbm.at[pl.ds(c * _NPAD + r0, _CHUNK)])


_BR = 1024


def _dinv_of(d0, d1):
    return lax.rsqrt(d0[:, 0:1] + d1[:, 0:1] + 1.0)


def _mm1_body(x_ref, w_ref, d0_ref, d1_ref, o_ref):
    dinv = _dinv_of(d0_ref[...], d1_ref[...])
    o_ref[...] = dinv * jnp.dot(x_ref[...], w_ref[...],
                                preferred_element_type=jnp.float32)


def _mm2_body(p0_ref, p1_ref, h1s_ref, d0_ref, d1_ref, w_ref, o_ref):
    dinv = _dinv_of(d0_ref[...], d1_ref[...])
    agg = dinv * (p0_ref[...] + p1_ref[...] + h1s_ref[...])
    h = jnp.maximum(agg, 0.0)
    o_ref[...] = dinv * jnp.dot(h, w_ref[...],
                                preferred_element_type=jnp.float32)


def _final_body(q0_ref, q1_ref, h2s_ref, d0_ref, d1_ref, o_ref):
    dinv = _dinv_of(d0_ref[...], d1_ref[...])
    agg = dinv * (q0_ref[...] + q1_ref[...] + h2s_ref[...])
    m = jnp.max(agg, axis=-1, keepdims=True)
    z = agg - m
    lse = jnp.log(jnp.sum(jnp.exp(z), axis=-1, keepdims=True))
    o_ref[...] = z - lse


def _row_spec(w):
    return pl.BlockSpec((_BR, w), lambda i: (i, 0))


def _full_spec(r, c):
    return pl.BlockSpec((r, c), lambda i: (0, 0))


def _mm1(x_pad, W1, d0, d1):
    return pl.pallas_call(
        _mm1_body,
        grid=(_NPAD // _BR,),
        in_specs=[_row_spec(_D), _full_spec(_D, _D), _row_spec(_D),
                  _row_spec(_D)],
        out_specs=_row_spec(_D),
        out_shape=jax.ShapeDtypeStruct((_NPAD, _D), jnp.float32),
    )(x_pad, W1, d0, d1)


def _mm2(p0, p1, h1s, d0, d1, W2):
    return pl.pallas_call(
        _mm2_body,
        grid=(_NPAD // _BR,),
        in_specs=[_row_spec(_D), _row_spec(_D), _row_spec(_D),
                  _row_spec(_D), _row_spec(_D), _full_spec(_D, _D)],
        out_specs=_row_spec(_D),
        out_shape=jax.ShapeDtypeStruct((_NPAD, _D), jnp.float32),
    )(p0, p1, h1s, d0, d1, W2)


def _final(q0, q1, h2s, d0, d1):
    return pl.pallas_call(
        _final_body,
        grid=(_NPAD // _BR,),
        in_specs=[_row_spec(_D), _row_spec(_D), _row_spec(_D),
                  _row_spec(_D), _row_spec(_D)],
        out_specs=_row_spec(_D),
        out_shape=jax.ShapeDtypeStruct((_NPAD, _D), jnp.float32),
    )(q0, q1, h2s, d0, d1)


def kernel(x, edge_index, W1, W2):
    src = edge_index[0]
    dst = edge_index[1]
    npadE = _ECHUNKS * _CHUNK - _E
    srcp = jnp.concatenate([src, jnp.zeros((npadE,), jnp.int32)])
    dst_flat = jnp.concatenate([dst, jnp.full((npadE,), _NPAD - 1, jnp.int32)])
    srcp = srcp.reshape(_ECHUNKS, _CHUNK)
    dstp = dst_flat.reshape(_ECHUNKS, _CHUNK)
    x_pad = jnp.pad(x, ((0, _NPAD - _N), (0, 0)))
    zeros128 = jnp.zeros((_CHUNK, _D), jnp.float32)
    ones_tab = jnp.ones((_NPAD, _D), jnp.float32)

    DP = _sc_spmm(ones_tab, srcp, dstp, zeros128)
    d0 = DP[:_NPAD]
    d1 = DP[_NPAD:]

    h1s = _mm1(x_pad, W1, d0, d1)
    P = _sc_spmm(h1s, srcp, dstp, zeros128)
    h2s = _mm2(P[:_NPAD], P[_NPAD:], h1s, d0, d1, W2)
    Q = _sc_spmm(h2s, srcp, dstp, zeros128)
    out = _final(Q[:_NPAD], Q[_NPAD:], h2s, d0, d1)
    return out[:_N]


# speedup vs baseline: 1.0349x; 1.0293x over previous
import functools

import jax
import jax.numpy as jnp
from jax import lax
from jax.experimental import pallas as pl
from jax.experimental.pallas import tpu as pltpu
from jax.experimental.pallas import tpu_sc as plsc

_N = 10000
_NPAD = 10240
_E = 320000
_D = 128
_CHUNK = 128
_NTILES = 32
_CPT = 80
_EPT = _CHUNK * _CPT
_EPAD = _NTILES * _EPT
_CPT0 = 136
_CPT1 = 24
_SEG = 56
_SEGS0 = (56, 56, 24)
_SEGS1 = (24, 0, 0)
_RPT = _NPAD // 16
_DEGW = 8
_NBUF = 5
_SLACK = 8
_ECHUNKS = _EPAD // _CHUNK + _SLACK

_mesh = plsc.VectorSubcoreMesh(core_axis_name="c", subcore_axis_name="s")


@functools.partial(
    pl.kernel,
    out_type=jax.ShapeDtypeStruct((2 * _NPAD, _D), jnp.float32),
    mesh=_mesh,
    scratch_types=[
        pltpu.VMEM((_SEG, _CHUNK), jnp.int32),
        pltpu.VMEM((_SEG, _CHUNK), jnp.int32),
        [pltpu.VMEM((_CHUNK, _D), jnp.float32)] * 2,
        [pltpu.SemaphoreType.DMA] * 2,
        pltpu.VMEM_SHARED((_NPAD, _D), jnp.float32),
    ],
)
def _sc_spmm(h_hbm, src_hbm, dst_hbm, zeros_hbm, out_hbm, src_all, dst_all,
             rows, gsem, shared_acc):
    c = lax.axis_index("c")
    s = lax.axis_index("s")

    pltpu.sync_copy(zeros_hbm, rows[0])
    for j in range(_RPT // _CHUNK):
        pltpu.sync_copy(rows[0],
                        shared_acc.at[pl.ds(s * _RPT + j * _CHUNK, _CHUNK)])
    plsc.subcore_barrier()

    def body(t, carry):
        i0 = t * 2
        i1 = t * 2 + 1
        pltpu.async_copy(h_hbm.at[src_all.at[i0]], rows[0], gsem[0]).wait()
        pltpu.async_copy(rows[0], shared_acc.at[dst_all.at[i0]], gsem[1],
                         add=True)
        pltpu.async_copy(h_hbm.at[src_all.at[i1]], rows[1], gsem[0]).wait()
        pltpu.make_async_copy(rows[0], shared_acc.at[dst_all.at[i0]],
                              gsem[1]).wait()
        pltpu.sync_copy(rows[1], shared_acc.at[dst_all.at[i1]], add=True)
        return carry

    tile_base = jnp.where(c == 0, s * _CPT0, 16 * _CPT0 + s * _CPT1)
    seg0 = 0
    for k in range(len(_SEGS0)):
        scnt = jnp.where(c == 0, _SEGS0[k], _SEGS1[k])
        off = jnp.where(c == 0, seg0, _SEGS1[:k] and sum(_SEGS1[:k]) or 0)
        base = tile_base + off
        pltpu.sync_copy(src_hbm.at[pl.ds(base, _SEG)], src_all)
        pltpu.sync_copy(dst_hbm.at[pl.ds(base, _SEG)], dst_all)
        lax.fori_loop(0, scnt // 2, body, 0)
        seg0 += _SEGS0[k]
    plsc.subcore_barrier()
    for j in range(_RPT // _CHUNK):
        r0 = s * _RPT + j * _CHUNK
        pltpu.sync_copy(shared_acc.at[pl.ds(r0, _CHUNK)], rows[0])
        pltpu.sync_copy(rows[0], out_hbm.at[pl.ds(c * _NPAD + r0, _CHUNK)])


_BR = 1024


def _dinv_of(d0, d1):
    return lax.rsqrt(d0[:, 0:1] + d1[:, 0:1] + 1.0)


def _mm1_body(x_ref, w_ref, d0_ref, d1_ref, o_ref):
    dinv = _dinv_of(d0_ref[...], d1_ref[...])
    o_ref[...] = dinv * jnp.dot(x_ref[...], w_ref[...],
                                preferred_element_type=jnp.float32)


def _mm2_body(p0_ref, p1_ref, h1s_ref, d0_ref, d1_ref, w_ref, o_ref):
    dinv = _dinv_of(d0_ref[...], d1_ref[...])
    agg = dinv * (p0_ref[...] + p1_ref[...] + h1s_ref[...])
    h = jnp.maximum(agg, 0.0)
    o_ref[...] = dinv * jnp.dot(h, w_ref[...],
                                preferred_element_type=jnp.float32)


def _final_body(q0_ref, q1_ref, h2s_ref, d0_ref, d1_ref, o_ref):
    dinv = _dinv_of(d0_ref[...], d1_ref[...])
    agg = dinv * (q0_ref[...] + q1_ref[...] + h2s_ref[...])
    m = jnp.max(agg, axis=-1, keepdims=True)
    z = agg - m
    lse = jnp.log(jnp.sum(jnp.exp(z), axis=-1, keepdims=True))
    o_ref[...] = z - lse


def _row_spec(w):
    return pl.BlockSpec((_BR, w), lambda i: (i, 0))


def _full_spec(r, c):
    return pl.BlockSpec((r, c), lambda i: (0, 0))


def _mm1(x_pad, W1, d0, d1):
    return pl.pallas_call(
        _mm1_body,
        grid=(_NPAD // _BR,),
        in_specs=[_row_spec(_D), _full_spec(_D, _D), _row_spec(_D),
                  _row_spec(_D)],
        out_specs=_row_spec(_D),
        out_shape=jax.ShapeDtypeStruct((_NPAD, _D), jnp.float32),
    )(x_pad, W1, d0, d1)


def _mm2(p0, p1, h1s, d0, d1, W2):
    return pl.pallas_call(
        _mm2_body,
        grid=(_NPAD // _BR,),
        in_specs=[_row_spec(_D), _row_spec(_D), _row_spec(_D),
                  _row_spec(_D), _row_spec(_D), _full_spec(_D, _D)],
        out_specs=_row_spec(_D),
        out_shape=jax.ShapeDtypeStruct((_NPAD, _D), jnp.float32),
    )(p0, p1, h1s, d0, d1, W2)


def _final(q0, q1, h2s, d0, d1):
    return pl.pallas_call(
        _final_body,
        grid=(_NPAD // _BR,),
        in_specs=[_row_spec(_D), _row_spec(_D), _row_spec(_D),
                  _row_spec(_D), _row_spec(_D)],
        out_specs=_row_spec(_D),
        out_shape=jax.ShapeDtypeStruct((_NPAD, _D), jnp.float32),
    )(q0, q1, h2s, d0, d1)


def kernel(x, edge_index, W1, W2):
    src = edge_index[0]
    dst = edge_index[1]
    npadE = _ECHUNKS * _CHUNK - _E
    srcp = jnp.concatenate([src, jnp.zeros((npadE,), jnp.int32)])
    dst_flat = jnp.concatenate([dst, jnp.full((npadE,), _NPAD - 1, jnp.int32)])
    srcp = srcp.reshape(_ECHUNKS, _CHUNK)
    dstp = dst_flat.reshape(_ECHUNKS, _CHUNK)
    x_pad = jnp.pad(x, ((0, _NPAD - _N), (0, 0)))
    zeros128 = jnp.zeros((_CHUNK, _D), jnp.float32)
    ones_tab = jnp.ones((_NPAD, _D), jnp.float32)

    DP = _sc_spmm(ones_tab, srcp, dstp, zeros128)
    d0 = DP[:_NPAD]
    d1 = DP[_NPAD:]

    h1s = _mm1(x_pad, W1, d0, d1)
    P = _sc_spmm(h1s, srcp, dstp, zeros128)
    h2s = _mm2(P[:_NPAD], P[_NPAD:], h1s, d0, d1, W2)
    Q = _sc_spmm(h2s, srcp, dstp, zeros128)
    out = _final(Q[:_NPAD], Q[_NPAD:], h2s, d0, d1)
    return out[:_N]
